# Initial kernel scaffold; baseline (speedup 1.0000x reference)
import jax
import jax.numpy as jnp
from jax.experimental import pallas as pl


def kernel(hidden_states, expert_routing_table, router_weights, w13, w2):
    raise NotImplementedError("write your pallas kernel here")


# speedup vs baseline: 1.0808x; 1.0808x over previous
import functools

import jax
import jax.numpy as jnp
from jax.experimental import pallas as pl
from jax.experimental.pallas import tpu as pltpu

E = 16
K = 2
T = 128
D = 1024
F = 2048

BF = 512
NJ = F // BF


def _moe_kernel(route_ref, rw_ref, x_ref, w1_ref, w3_ref, w2_ref, out_ref):
    e = pl.program_id(0)
    j = pl.program_id(1)

    @pl.when(jnp.logical_and(e == 0, j == 0))
    def _():
        out_ref[:, :] = jnp.zeros_like(out_ref)

    x = x_ref[:, :]
    w1 = w1_ref[0, 0]
    w3 = w3_ref[0, 0]
    w2 = w2_ref[0]

    g = jax.lax.dot_general(x, w1, (((1,), (1,)), ((), ())))
    u = jax.lax.dot_general(x, w3, (((1,), (1,)), ((), ())))
    h = (g * jax.nn.sigmoid(g)) * u
    y = jax.lax.dot_general(h, w2, (((1,), (1,)), ((), ())))

    sel = (route_ref[:, :] == e).astype(jnp.float32)
    combine = jnp.sum(sel * rw_ref[:, :], axis=1, keepdims=True)

    out_ref[:, :] += combine * y


@jax.jit
def kernel(hidden_states, expert_routing_table, router_weights, w13, w2):
    route = expert_routing_table.astype(jnp.int32)
    w13r = w13.reshape(E, 2, F, D)

    out = pl.pallas_call(
        _moe_kernel,
        grid=(E, NJ),
        in_specs=[
            pl.BlockSpec((T, K), lambda e, j: (0, 0)),
            pl.BlockSpec((T, K), lambda e, j: (0, 0)),
            pl.BlockSpec((T, D), lambda e, j: (0, 0)),
            pl.BlockSpec((1, 1, BF, D), lambda e, j: (e, 0, j, 0)),
            pl.BlockSpec((1, 1, BF, D), lambda e, j: (e, 1, j, 0)),
            pl.BlockSpec((1, D, BF), lambda e, j: (e, 0, j)),
        ],
        out_specs=pl.BlockSpec((T, D), lambda e, j: (0, 0)),
        out_shape=jax.ShapeDtypeStruct((T, D), jnp.float32),
        compiler_params=pltpu.CompilerParams(
            dimension_semantics=("arbitrary", "arbitrary"),
        ),
    )(route, router_weights, hidden_states, w13r, w13r, w2)
    return out


# speedup vs baseline: 1.2007x; 1.1110x over previous
import functools

import jax
import jax.numpy as jnp
from jax.experimental import pallas as pl
from jax.experimental.pallas import tpu as pltpu

E = 16
K = 2
T = 128
D = 1024
F = 2048

BF = 1024
NJ = F // BF


def _moe_kernel(route_ref, rw_ref, x_ref, w1_ref, w3_ref, w2_ref, out_ref):
    e = pl.program_id(0)
    j = pl.program_id(1)

    @pl.when(jnp.logical_and(e == 0, j == 0))
    def _():
        out_ref[:, :] = jnp.zeros_like(out_ref)

    x = x_ref[:, :]
    w1 = w1_ref[0, 0]
    w3 = w3_ref[0, 0]
    w2 = w2_ref[0]

    g = jax.lax.dot_general(x, w1, (((1,), (1,)), ((), ())))
    u = jax.lax.dot_general(x, w3, (((1,), (1,)), ((), ())))
    h = (g * jax.nn.sigmoid(g)) * u
    y = jax.lax.dot_general(h, w2, (((1,), (1,)), ((), ())))

    sel = (route_ref[:, :] == e).astype(jnp.float32)
    combine = jnp.sum(sel * rw_ref[:, :], axis=1, keepdims=True)

    out_ref[:, :] += combine * y


@jax.jit
def kernel(hidden_states, expert_routing_table, router_weights, w13, w2):
    route = expert_routing_table.astype(jnp.int32)
    w13r = w13.reshape(E, 2, F, D)

    out = pl.pallas_call(
        _moe_kernel,
        grid=(E, NJ),
        in_specs=[
            pl.BlockSpec((T, K), lambda e, j: (0, 0)),
            pl.BlockSpec((T, K), lambda e, j: (0, 0)),
            pl.BlockSpec((T, D), lambda e, j: (0, 0)),
            pl.BlockSpec((1, 1, BF, D), lambda e, j: (e, 0, j, 0)),
            pl.BlockSpec((1, 1, BF, D), lambda e, j: (e, 1, j, 0)),
            pl.BlockSpec((1, D, BF), lambda e, j: (e, 0, j)),
        ],
        out_specs=pl.BlockSpec((T, D), lambda e, j: (0, 0)),
        out_shape=jax.ShapeDtypeStruct((T, D), jnp.float32),
        compiler_params=pltpu.CompilerParams(
            dimension_semantics=("arbitrary", "arbitrary"),
        ),
    )(route, router_weights, hidden_states, w13r, w13r, w2)
    return out
